# baseline (speedup 1.0000x reference)
import functools

import jax
import jax.numpy as jnp
from jax import lax
from jax.experimental import pallas as pl
from jax.experimental.pallas import tpu as pltpu
from jax.experimental.pallas import tpu_sc as plsc

NUM_ENTITIES = 1000000
EMBED_DIM = 64
BATCH = 16384

_info = plsc.get_sparse_core_info()
_NC, _NS = _info.num_cores, _info.num_subcores
_NW = _NC * _NS
_B_PER_W = BATCH // _NW

_mesh = plsc.VectorSubcoreMesh(core_axis_name="c", subcore_axis_name="s")


@functools.partial(
    pl.kernel,
    mesh=_mesh,
    out_type=jax.ShapeDtypeStruct((EMBED_DIM, BATCH), jnp.float32),
    scratch_types=[
        pltpu.VMEM((_B_PER_W,), jnp.int32),
        pltpu.VMEM((EMBED_DIM * _B_PER_W,), jnp.int32),
        pltpu.VMEM((EMBED_DIM * _B_PER_W,), jnp.float32),
        pltpu.SemaphoreType.DMA,
    ],
)
def _gather_kernel(idx_hbm, table_hbm, out_hbm, idx_v, idx2, buf, sem):
    wid = lax.axis_index("s") * _NC + lax.axis_index("c")
    base = wid * _B_PER_W
    pltpu.sync_copy(idx_hbm.at[pl.ds(base, _B_PER_W)], idx_v)

    def build_body(j, carry):
        off = j * NUM_ENTITIES
        for g in range(_B_PER_W // 16):
            v = idx_v[pl.ds(g * 16, 16)]
            idx2[pl.ds(j * _B_PER_W + g * 16, 16)] = v + off
        return carry

    lax.fori_loop(0, EMBED_DIM, build_body, 0)
    cp = pltpu.make_async_copy(table_hbm.at[idx2], buf, sem)
    cp.start()
    cp.wait()
    for j in range(EMBED_DIM):
        pltpu.sync_copy(
            buf.at[pl.ds(j * _B_PER_W, _B_PER_W)],
            out_hbm.at[j, pl.ds(base, _B_PER_W)],
        )


def kernel(entities, entity_embeddings):
    flat_t = entity_embeddings.T.reshape(NUM_ENTITIES * EMBED_DIM)
    out_t = _gather_kernel(entities, flat_t)
    return out_t.T
